# baseline (speedup 1.0000x reference)
import functools
import jax
import jax.numpy as jnp
from jax import lax
from jax.experimental import pallas as pl
from jax.experimental.pallas import tpu as pltpu, tpu_sc as plsc

N = 10000
E = 160000
F = 128
NP = 10240
EP = 163840
G = 64
H = 4
A = 16
NCLS = 10
CHUNK = 128
NSUB = 16
NCORE = 2
NWORK = NCORE * NSUB
CH_PER_W = EP // (CHUNK * NWORK)
ROWS_PER_SUB = NP // NSUB
BN = 1024
NBLK = NP // BN

def _get_mesh():
    return plsc.VectorSubcoreMesh(core_axis_name="c", subcore_axis_name="s",
                                  num_cores=NCORE, num_subcores=NSUB)


def _sc_agg_deg(x_hbm, srcp, dstp, z2d, z1d, ones_hbm):
    @functools.partial(
        pl.kernel,
        out_type=(jax.ShapeDtypeStruct((NCORE, NP, F), jnp.float32),
                  jax.ShapeDtypeStruct((NCORE, NP), jnp.float32)),
        mesh=_get_mesh(),
        scratch_types=[
            pltpu.VMEM((CHUNK,), jnp.int32),
            pltpu.VMEM((CHUNK,), jnp.int32),
            pltpu.VMEM((CHUNK, F), jnp.float32),
            pltpu.VMEM((CHUNK,), jnp.float32),
            pltpu.VMEM_SHARED((NP, F), jnp.float32),
            pltpu.VMEM_SHARED((NP,), jnp.float32),
            pltpu.SemaphoreType.DMA,
        ],
    )
    def k(x_r, src_r, dst_r, z2d_r, z1d_r, ones_r, out_r, outd_r,
          idx_s, idx_d, rows, ones_v, acc, accd, sem):
        c = lax.axis_index("c")
        s = lax.axis_index("s")
        w = s * NCORE + c
        r0 = s * ROWS_PER_SUB
        pltpu.sync_copy(z2d_r, acc.at[pl.ds(r0, ROWS_PER_SUB)])
        pltpu.sync_copy(z1d_r, accd.at[pl.ds(r0, ROWS_PER_SUB)])
        pltpu.sync_copy(ones_r, ones_v)
        plsc.subcore_barrier()
        base = w * (CH_PER_W * CHUNK)

        def step(t, _):
            off = base + t * CHUNK
            pltpu.sync_copy(src_r.at[pl.ds(off, CHUNK)], idx_s)
            pltpu.sync_copy(dst_r.at[pl.ds(off, CHUNK)], idx_d)
            pltpu.async_copy(x_r.at[idx_s], rows, sem).wait()
            pltpu.sync_copy(rows, acc.at[idx_d], add=True)
            pltpu.sync_copy(ones_v, accd.at[idx_d], add=True)
            return _

        lax.fori_loop(0, CH_PER_W, step, None)
        plsc.subcore_barrier()
        pltpu.sync_copy(acc.at[pl.ds(r0, ROWS_PER_SUB)],
                        out_r.at[c, pl.ds(r0, ROWS_PER_SUB)])
        pltpu.sync_copy(accd.at[pl.ds(r0, ROWS_PER_SUB)],
                        outd_r.at[c, pl.ds(r0, ROWS_PER_SUB)])

    return k(x_hbm, srcp, dstp, z2d, z1d, ones_hbm)


def _sc_agg(x_hbm, srcp, dstp, z2d):
    @functools.partial(
        pl.kernel,
        out_type=jax.ShapeDtypeStruct((NCORE, NP, F), jnp.float32),
        mesh=_get_mesh(),
        scratch_types=[
            pltpu.VMEM((CHUNK,), jnp.int32),
            pltpu.VMEM((CHUNK,), jnp.int32),
            pltpu.VMEM((CHUNK, F), jnp.float32),
            pltpu.VMEM_SHARED((NP, F), jnp.float32),
            pltpu.SemaphoreType.DMA,
        ],
    )
    def k(x_r, src_r, dst_r, z2d_r, out_r, idx_s, idx_d, rows, acc, sem):
        c = lax.axis_index("c")
        s = lax.axis_index("s")
        w = s * NCORE + c
        r0 = s * ROWS_PER_SUB
        pltpu.sync_copy(z2d_r, acc.at[pl.ds(r0, ROWS_PER_SUB)])
        plsc.subcore_barrier()
        base = w * (CH_PER_W * CHUNK)

        def step(t, _):
            off = base + t * CHUNK
            pltpu.sync_copy(src_r.at[pl.ds(off, CHUNK)], idx_s)
            pltpu.sync_copy(dst_r.at[pl.ds(off, CHUNK)], idx_d)
            pltpu.async_copy(x_r.at[idx_s], rows, sem).wait()
            pltpu.sync_copy(rows, acc.at[idx_d], add=True)
            return _

        lax.fori_loop(0, CH_PER_W, step, None)
        plsc.subcore_barrier()
        pltpu.sync_copy(acc.at[pl.ds(r0, ROWS_PER_SUB)],
                        out_r.at[c, pl.ds(r0, ROWS_PER_SUB)])

    return k(x_hbm, srcp, dstp, z2d)


def _sc_mask(assign, srcp, dstp):
    @functools.partial(
        pl.kernel,
        out_type=(jax.ShapeDtypeStruct((EP,), jnp.int32),
                  jax.ShapeDtypeStruct((EP,), jnp.int32)),
        mesh=_get_mesh(),
        scratch_types=[
            pltpu.VMEM((CHUNK,), jnp.int32),
            pltpu.VMEM((CHUNK,), jnp.int32),
            pltpu.VMEM((CHUNK,), jnp.int32),
            pltpu.VMEM((CHUNK,), jnp.int32),
            pltpu.VMEM((CHUNK,), jnp.int32),
            pltpu.VMEM((CHUNK,), jnp.int32),
            pltpu.SemaphoreType.DMA,
        ],
    )
    def k(asg_r, src_r, dst_r, src2_r, dst2_r, bs, bd, a1, a2, bs2, bd2, sem):
        c = lax.axis_index("c")
        s = lax.axis_index("s")
        w = s * NCORE + c
        base = w * (CH_PER_W * CHUNK)
        dummy = jnp.full((16,), N, jnp.int32)

        def step(t, _):
            off = base + t * CHUNK
            pltpu.sync_copy(src_r.at[pl.ds(off, CHUNK)], bs)
            pltpu.sync_copy(dst_r.at[pl.ds(off, CHUNK)], bd)
            cp1 = pltpu.async_copy(asg_r.at[bs], a1, sem)
            cp2 = pltpu.async_copy(asg_r.at[bd], a2, sem)
            cp1.wait()
            cp2.wait()
            for j in range(CHUNK // 16):
                si = bs[pl.ds(j * 16, 16)]
                di = bd[pl.ds(j * 16, 16)]
                eq = a1[pl.ds(j * 16, 16)] == a2[pl.ds(j * 16, 16)]
                bs2[pl.ds(j * 16, 16)] = jnp.where(eq, si, dummy)
                bd2[pl.ds(j * 16, 16)] = jnp.where(eq, di, dummy)
            pltpu.sync_copy(bs2, src2_r.at[pl.ds(off, CHUNK)])
            pltpu.sync_copy(bd2, dst2_r.at[pl.ds(off, CHUNK)])
            return _

        lax.fori_loop(0, CH_PER_W, step, None)

    return k(assign, srcp, dstp)


def _rowspec(bn=BN, w=F):
    return pl.BlockSpec((bn, w), lambda i: (i, 0))


def _fullspec(shape):
    nd = len(shape)
    return pl.BlockSpec(shape, lambda i: (0,) * nd)


def _tc_enc1(x, p0, p1, d0, d1, W, b):
    def body(x_r, p0_r, p1_r, d0_r, d1_r, W_r, b_r, h_r, inv_r):
        deg = jnp.maximum(d0_r[...] + d1_r[...], 1.0)
        inv = 1.0 / deg
        agg = (p0_r[...] + p1_r[...]) * inv
        h_r[...] = jnp.maximum(
            jnp.dot(x_r[...] + agg, W_r[...],
                    preferred_element_type=jnp.float32) + b_r[...], 0.0)
        inv_r[...] = inv

    return pl.pallas_call(
        body,
        grid=(NBLK,),
        in_specs=[_rowspec(), _rowspec(), _rowspec(),
                  _rowspec(w=1), _rowspec(w=1),
                  _fullspec((F, F)), _fullspec((1, F))],
        out_specs=[_rowspec(), _rowspec(w=1)],
        out_shape=[jax.ShapeDtypeStruct((NP, F), jnp.float32),
                   jax.ShapeDtypeStruct((NP, 1), jnp.float32)],
    )(x, p0, p1, d0, d1, W, b)


def _tc_enc2(h1, q0, q1, inv, W, b, dTp):
    def body(h1_r, q0_r, q1_r, inv_r, W_r, b_r, dT_r, h_r, asg_r):
        agg = (q0_r[...] + q1_r[...]) * inv_r[...]
        h = jnp.maximum(
            jnp.dot(h1_r[...] + agg, W_r[...],
                    preferred_element_type=jnp.float32) + b_r[...], 0.0)
        h_r[...] = h
        lg = jnp.dot(h, dT_r[...], preferred_element_type=jnp.float32)
        col = lax.broadcasted_iota(jnp.int32, (BN, F), 1)
        lgm = jnp.where(col < A, lg, -1e30)
        m = jnp.max(lgm, axis=1, keepdims=True)
        asg_r[...] = jnp.min(jnp.where(lgm >= m, col, F), axis=1, keepdims=True)

    return pl.pallas_call(
        body,
        grid=(NBLK,),
        in_specs=[_rowspec(), _rowspec(), _rowspec(), _rowspec(w=1),
                  _fullspec((F, F)), _fullspec((1, F)), _fullspec((F, F))],
        out_specs=[_rowspec(), _rowspec(w=1)],
        out_shape=[jax.ShapeDtypeStruct((NP, F), jnp.float32),
                   jax.ShapeDtypeStruct((NP, 1), jnp.int32)],
    )(h1, q0, q1, inv, W, b, dTp)


def _tc_gconv(xin, r0, r1, Wself, Wk, b):
    def body(x_r, r0_r, r1_r, Ws_r, Wk_r, b_r, out_r):
        aggk = r0_r[...] + r1_r[...]
        Wk = Wk_r[...]
        ek = (jnp.dot(aggk, Wk[0], preferred_element_type=jnp.float32)
              + jnp.dot(aggk, Wk[1], preferred_element_type=jnp.float32)
              + jnp.dot(aggk, Wk[2], preferred_element_type=jnp.float32))
        out_r[...] = jnp.maximum(
            jnp.dot(x_r[...], Ws_r[...], preferred_element_type=jnp.float32)
            + ek * (1.0 / 3.0) + b_r[...], 0.0)

    return pl.pallas_call(
        body,
        grid=(NBLK,),
        in_specs=[_rowspec(), _rowspec(), _rowspec(),
                  _fullspec((F, F)), _fullspec((3, F, F)), _fullspec((1, F))],
        out_specs=_rowspec(),
        out_shape=jax.ShapeDtypeStruct((NP, F), jnp.float32),
    )(xin, r0, r1, Wself, Wk, b)


def _tc_pool1(x1, x2, x3, pp, awTp, batch2d):
    def body(x1_r, x2_r, x3_r, pp_r, aw_r, b_r, xc_r, gate_r, gm_r, mscr):
        i = pl.program_id(0)
        pp = pp_r[...]
        pn = pp * (1.0 / (jnp.sqrt(jnp.sum(pp * pp)) + 1e-9))
        x1v, x2v, x3v = x1_r[...], x2_r[...], x3_r[...]
        hi = lax.Precision.HIGHEST
        s1 = jnp.tanh(jnp.dot(x1v, pn, precision=hi,
                              preferred_element_type=jnp.float32))
        s2 = jnp.tanh(jnp.dot(x2v, pn, precision=hi,
                              preferred_element_type=jnp.float32))
        s3 = jnp.tanh(jnp.dot(x3v, pn, precision=hi,
                              preferred_element_type=jnp.float32))
        m = jnp.maximum(jnp.maximum(s1, s2), s3)
        e1 = jnp.exp(s1 - m)
        e2 = jnp.exp(s2 - m)
        e3 = jnp.exp(s3 - m)
        es = e1 + e2 + e3
        xc = (e1 * x1v + e2 * x2v + e3 * x3v) / es
        xc_r[...] = xc
        gate = jnp.dot(xc, aw_r[...], preferred_element_type=jnp.float32)
        row = lax.broadcasted_iota(jnp.int32, (BN, F), 0) + i * BN
        col = lax.broadcasted_iota(jnp.int32, (BN, F), 1)
        gate = jnp.where((row < N) & (col < H), gate, -1e30)
        gate_r[...] = gate

        @pl.when(i == 0)
        def _():
            mscr[...] = jnp.full((8, G), -1e30, jnp.float32)

        oh = b_r[...] == lax.broadcasted_iota(jnp.int32, (BN, G), 1)
        for h in range(H):
            tmp = jnp.where(oh, jnp.broadcast_to(gate[:, h:h + 1], (BN, G)),
                            -1e30)
            mh = jnp.max(tmp, axis=0, keepdims=True)
            mscr[h:h + 1, :] = jnp.maximum(mscr[h:h + 1, :], mh)
        gm_r[...] = mscr[...]

    return pl.pallas_call(
        body,
        grid=(NBLK,),
        in_specs=[_rowspec(), _rowspec(), _rowspec(),
                  _fullspec((F, 1)), _fullspec((F, F)), _rowspec(w=1)],
        out_specs=[_rowspec(), _rowspec(), pl.BlockSpec((8, G), lambda i: (0, 0))],
        out_shape=[jax.ShapeDtypeStruct((NP, F), jnp.float32),
                   jax.ShapeDtypeStruct((NP, F), jnp.float32),
                   jax.ShapeDtypeStruct((8, G), jnp.float32)],
        scratch_shapes=[pltpu.VMEM((8, G), jnp.float32)],
    )(x1, x2, x3, pp, awTp, batch2d)


def _tc_pool2(gate, gmaxp, batch2d):
    def body(gate_r, gm_r, b_r, eg_r, den_r, accden):
        i = pl.program_id(0)
        oh = (b_r[...] == lax.broadcasted_iota(jnp.int32, (BN, G), 1)
              ).astype(jnp.float32)
        gmb = jnp.dot(oh, gm_r[...], precision=lax.Precision.HIGHEST,
                      preferred_element_type=jnp.float32)
        eg = jnp.exp(gate_r[...] - gmb)
        eg_r[...] = eg

        @pl.when(i == 0)
        def _():
            accden[...] = jnp.zeros((G, F), jnp.float32)

        accden[...] += lax.dot_general(oh, eg, (((0,), (0,)), ((), ())),
                                       precision=lax.Precision.HIGHEST,
                                       preferred_element_type=jnp.float32)
        den_r[...] = accden[...]

    return pl.pallas_call(
        body,
        grid=(NBLK,),
        in_specs=[_rowspec(), pl.BlockSpec((G, F), lambda i: (0, 0)),
                  _rowspec(w=1)],
        out_specs=[_rowspec(), pl.BlockSpec((G, F), lambda i: (0, 0))],
        out_shape=[jax.ShapeDtypeStruct((NP, F), jnp.float32),
                   jax.ShapeDtypeStruct((G, F), jnp.float32)],
        scratch_shapes=[pltpu.VMEM((G, F), jnp.float32)],
    )(gate, gmaxp, batch2d)


def _tc_pool3(eg, den, xc, batch2d, W1, b1, W2p, b2p):
    def body(eg_r, den_r, xc_r, b_r, W1_r, b1_r, W2_r, b2_r,
             gf_r, lg_r, accgf):
        i = pl.program_id(0)
        oh = (b_r[...] == lax.broadcasted_iota(jnp.int32, (BN, G), 1)
              ).astype(jnp.float32)
        denb = jnp.dot(oh, den_r[...], precision=lax.Precision.HIGHEST,
                       preferred_element_type=jnp.float32)
        att = eg_r[...] / (denb + 1e-9)
        attbar = jnp.sum(att, axis=1, keepdims=True) * (1.0 / H)

        @pl.when(i == 0)
        def _():
            accgf[...] = jnp.zeros((G, F), jnp.float32)

        accgf[...] += lax.dot_general(oh, attbar * xc_r[...],
                                      (((0,), (0,)), ((), ())),
                                      precision=lax.Precision.HIGHEST,
                                      preferred_element_type=jnp.float32)

        @pl.when(i == NBLK - 1)
        def _():
            gf = accgf[...]
            gf_r[...] = gf
            z = jnp.maximum(
                jnp.dot(gf, W1_r[...], preferred_element_type=jnp.float32)
                + b1_r[...], 0.0)
            u = jnp.dot(z, W2_r[...], preferred_element_type=jnp.float32) \
                + b2_r[...]
            col = lax.broadcasted_iota(jnp.int32, (G, F), 1)
            um = jnp.where(col < NCLS, u, -1e30)
            m = jnp.max(um, axis=1, keepdims=True)
            esum = jnp.sum(jnp.where(col < NCLS, jnp.exp(um - m), 0.0),
                           axis=1, keepdims=True)
            lg_r[...] = jnp.where(col < NCLS, u - m - jnp.log(esum), 0.0)

    return pl.pallas_call(
        body,
        grid=(NBLK,),
        in_specs=[_rowspec(), pl.BlockSpec((G, F), lambda i: (0, 0)),
                  _rowspec(), _rowspec(w=1),
                  _fullspec((F, F)), _fullspec((1, F)),
                  _fullspec((F, F)), _fullspec((1, F))],
        out_specs=[pl.BlockSpec((G, F), lambda i: (0, 0)),
                   pl.BlockSpec((G, F), lambda i: (0, 0))],
        out_shape=[jax.ShapeDtypeStruct((G, F), jnp.float32),
                   jax.ShapeDtypeStruct((G, F), jnp.float32)],
        scratch_shapes=[pltpu.VMEM((G, F), jnp.float32)],
    )(eg, den, xc, batch2d, W1, b1, W2p, b2p)


def kernel(x, edge_index, batch, enc_W1, enc_b1, enc_W2, enc_b2, dictionary,
           g1_Wself, g1_Wk, g1_b, g2_Wself, g2_Wk, g2_b, g3_Wself, g3_Wk, g3_b,
           pool_p, att_w, cls_W1, cls_b1, cls_W2, cls_b2):
    f32 = jnp.float32
    x_ext = jnp.concatenate([x, jnp.zeros((NP - N, F), f32)], axis=0)
    padi = jnp.full((EP - E,), N, jnp.int32)
    srcp = jnp.concatenate([edge_index[0], padi])
    dstp = jnp.concatenate([edge_index[1], padi])
    z2d = jnp.zeros((ROWS_PER_SUB, F), f32)
    z1d = jnp.zeros((ROWS_PER_SUB,), f32)
    ones_c = jnp.ones((CHUNK,), f32)

    part, degp = _sc_agg_deg(x_ext, srcp, dstp, z2d, z1d, ones_c)
    d0 = degp[0].reshape(NP, 1)
    d1 = degp[1].reshape(NP, 1)
    h1, inv = _tc_enc1(x_ext, part[0], part[1], d0, d1,
                       enc_W1, enc_b1.reshape(1, F))

    q = _sc_agg(h1, srcp, dstp, z2d)
    dTp = jnp.zeros((F, F), f32).at[:, :A].set(dictionary.T)
    h2, asg = _tc_enc2(h1, q[0], q[1], inv, enc_W2, enc_b2.reshape(1, F), dTp)
    src2, dst2 = _sc_mask(asg.reshape(NP), srcp, dstp)

    r = _sc_agg(h2, src2, dst2, z2d)
    x1 = _tc_gconv(h2, r[0], r[1], g1_Wself, g1_Wk, g1_b.reshape(1, F))
    r = _sc_agg(x1, src2, dst2, z2d)
    x2 = _tc_gconv(x1, r[0], r[1], g2_Wself, g2_Wk, g2_b.reshape(1, F))
    r = _sc_agg(x2, src2, dst2, z2d)
    x3 = _tc_gconv(x2, r[0], r[1], g3_Wself, g3_Wk, g3_b.reshape(1, F))

    awTp = jnp.zeros((F, F), f32).at[:, :H].set(att_w.T)
    batch2d = jnp.concatenate([batch, jnp.full((NP - N,), G - 1, jnp.int32)]
                              ).reshape(NP, 1)
    xc, gate, gmax8 = _tc_pool1(x1, x2, x3, pool_p.reshape(F, 1), awTp, batch2d)

    gm4 = gmax8[:H]
    gm4 = jnp.where(gm4 > -1e29, gm4, 0.0)
    gmaxp = jnp.zeros((G, F), f32).at[:, :H].set(gm4.T)
    eg, den = _tc_pool2(gate, gmaxp, batch2d)

    W2p = jnp.zeros((F, F), f32).at[:, :NCLS].set(cls_W2)
    b2p = jnp.zeros((1, F), f32).at[:, :NCLS].set(cls_b2)
    gf, lgfull = _tc_pool3(eg, den, xc, batch2d,
                           cls_W1, cls_b1.reshape(1, F), W2p, b2p)
    return (lgfull[:, :NCLS], gf)


# speedup vs baseline: 9.0318x; 9.0318x over previous
import functools
import jax
import jax.numpy as jnp
from jax import lax
from jax.experimental import pallas as pl
from jax.experimental.pallas import tpu as pltpu, tpu_sc as plsc

N = 10000
E = 160000
F = 128
NP = 10240
EP = 163840
G = 64
H = 4
A = 16
NCLS = 10
CHUNK = 128
NSUB = 16
NCORE = 2
NWORK = NCORE * NSUB
CH_PER_W = EP // (CHUNK * NWORK)
ROWS_PER_SUB = NP // NSUB
BN = 1024
NBLK = NP // BN

def _get_mesh():
    return plsc.VectorSubcoreMesh(core_axis_name="c", subcore_axis_name="s",
                                  num_cores=NCORE, num_subcores=NSUB)


def _sc_agg_deg(x_hbm, srcp, dstp, z2d, z1d, ones_hbm):
    @functools.partial(
        pl.kernel,
        out_type=(jax.ShapeDtypeStruct((NCORE, NP, F), jnp.float32),
                  jax.ShapeDtypeStruct((NCORE, NP), jnp.float32)),
        mesh=_get_mesh(),
        scratch_types=[
            pltpu.VMEM((CHUNK,), jnp.int32),
            pltpu.VMEM((CHUNK,), jnp.int32),
            pltpu.VMEM((CHUNK, F), jnp.float32),
            pltpu.VMEM((CHUNK,), jnp.float32),
            pltpu.VMEM_SHARED((NP, F), jnp.float32),
            pltpu.VMEM_SHARED((NP,), jnp.float32),
            pltpu.SemaphoreType.DMA,
        ],
    )
    def k(x_r, src_r, dst_r, z2d_r, z1d_r, ones_r, out_r, outd_r,
          idx_s, idx_d, rows, ones_v, acc, accd, sem):
        c = lax.axis_index("c")
        s = lax.axis_index("s")
        w = s * NCORE + c
        r0 = s * ROWS_PER_SUB
        pltpu.sync_copy(z2d_r, acc.at[pl.ds(r0, ROWS_PER_SUB)])
        pltpu.sync_copy(z1d_r, accd.at[pl.ds(r0, ROWS_PER_SUB)])
        pltpu.sync_copy(ones_r, ones_v)
        plsc.subcore_barrier()
        base = w * (CH_PER_W * CHUNK)

        def step(t, _):
            off = base + t * CHUNK
            pltpu.sync_copy(src_r.at[pl.ds(off, CHUNK)], idx_s)
            pltpu.sync_copy(dst_r.at[pl.ds(off, CHUNK)], idx_d)
            pltpu.async_copy(x_r.at[idx_s], rows, sem).wait()
            pltpu.sync_copy(rows, acc.at[idx_d], add=True)
            pltpu.sync_copy(ones_v, accd.at[idx_d], add=True)
            return _

        lax.fori_loop(0, CH_PER_W, step, None)
        plsc.subcore_barrier()
        pltpu.sync_copy(acc.at[pl.ds(r0, ROWS_PER_SUB)],
                        out_r.at[c, pl.ds(r0, ROWS_PER_SUB)])
        pltpu.sync_copy(accd.at[pl.ds(r0, ROWS_PER_SUB)],
                        outd_r.at[c, pl.ds(r0, ROWS_PER_SUB)])

    return k(x_hbm, srcp, dstp, z2d, z1d, ones_hbm)


def _sc_agg(x_hbm, srcp, dstp, z2d):
    @functools.partial(
        pl.kernel,
        out_type=jax.ShapeDtypeStruct((NCORE, NP, F), jnp.float32),
        mesh=_get_mesh(),
        scratch_types=[
            pltpu.VMEM((CHUNK,), jnp.int32),
            pltpu.VMEM((CHUNK,), jnp.int32),
            pltpu.VMEM((CHUNK, F), jnp.float32),
            pltpu.VMEM_SHARED((NP, F), jnp.float32),
            pltpu.SemaphoreType.DMA,
        ],
    )
    def k(x_r, src_r, dst_r, z2d_r, out_r, idx_s, idx_d, rows, acc, sem):
        c = lax.axis_index("c")
        s = lax.axis_index("s")
        w = s * NCORE + c
        r0 = s * ROWS_PER_SUB
        pltpu.sync_copy(z2d_r, acc.at[pl.ds(r0, ROWS_PER_SUB)])
        plsc.subcore_barrier()
        base = w * (CH_PER_W * CHUNK)

        def step(t, _):
            off = base + t * CHUNK
            pltpu.sync_copy(src_r.at[pl.ds(off, CHUNK)], idx_s)
            pltpu.sync_copy(dst_r.at[pl.ds(off, CHUNK)], idx_d)
            pltpu.async_copy(x_r.at[idx_s], rows, sem).wait()
            pltpu.sync_copy(rows, acc.at[idx_d], add=True)
            return _

        lax.fori_loop(0, CH_PER_W, step, None)
        plsc.subcore_barrier()
        pltpu.sync_copy(acc.at[pl.ds(r0, ROWS_PER_SUB)],
                        out_r.at[c, pl.ds(r0, ROWS_PER_SUB)])

    return k(x_hbm, srcp, dstp, z2d)


def _sc_mask(assign, srcp, dstp):
    @functools.partial(
        pl.kernel,
        out_type=(jax.ShapeDtypeStruct((EP,), jnp.int32),
                  jax.ShapeDtypeStruct((EP,), jnp.int32)),
        mesh=_get_mesh(),
        scratch_types=[
            pltpu.VMEM((CHUNK,), jnp.int32),
            pltpu.VMEM((CHUNK,), jnp.int32),
            pltpu.VMEM((CHUNK,), jnp.int32),
            pltpu.VMEM((CHUNK,), jnp.int32),
            pltpu.VMEM((CHUNK,), jnp.int32),
            pltpu.VMEM((CHUNK,), jnp.int32),
            pltpu.SemaphoreType.DMA,
        ],
    )
    def k(asg_r, src_r, dst_r, src2_r, dst2_r, bs, bd, a1, a2, bs2, bd2, sem):
        c = lax.axis_index("c")
        s = lax.axis_index("s")
        w = s * NCORE + c
        base = w * (CH_PER_W * CHUNK)
        lane = jax.lax.iota(jnp.int32, 16)

        def step(t, _):
            off = base + t * CHUNK
            pltpu.sync_copy(src_r.at[pl.ds(off, CHUNK)], bs)
            pltpu.sync_copy(dst_r.at[pl.ds(off, CHUNK)], bd)
            cp1 = pltpu.async_copy(asg_r.at[bs], a1, sem)
            cp2 = pltpu.async_copy(asg_r.at[bd], a2, sem)
            cp1.wait()
            cp2.wait()
            for j in range(CHUNK // 16):
                si = bs[pl.ds(j * 16, 16)]
                di = bd[pl.ds(j * 16, 16)]
                eq = a1[pl.ds(j * 16, 16)] == a2[pl.ds(j * 16, 16)]
                dummy = N + jnp.remainder(off + j * 16 + lane, NP - N)
                bs2[pl.ds(j * 16, 16)] = jnp.where(eq, si, dummy)
                bd2[pl.ds(j * 16, 16)] = jnp.where(eq, di, dummy)
            pltpu.sync_copy(bs2, src2_r.at[pl.ds(off, CHUNK)])
            pltpu.sync_copy(bd2, dst2_r.at[pl.ds(off, CHUNK)])
            return _

        lax.fori_loop(0, CH_PER_W, step, None)

    return k(assign, srcp, dstp)


def _rowspec(bn=BN, w=F):
    return pl.BlockSpec((bn, w), lambda i: (i, 0))


def _fullspec(shape):
    nd = len(shape)
    return pl.BlockSpec(shape, lambda i: (0,) * nd)


def _tc_enc1(x, p0, p1, d0, d1, W, b):
    def body(x_r, p0_r, p1_r, d0_r, d1_r, W_r, b_r, h_r, inv_r):
        deg = jnp.maximum(d0_r[...] + d1_r[...], 1.0)
        inv = 1.0 / deg
        agg = (p0_r[...] + p1_r[...]) * inv
        h_r[...] = jnp.maximum(
            jnp.dot(x_r[...] + agg, W_r[...],
                    preferred_element_type=jnp.float32) + b_r[...], 0.0)
        inv_r[...] = inv

    return pl.pallas_call(
        body,
        grid=(NBLK,),
        in_specs=[_rowspec(), _rowspec(), _rowspec(),
                  _rowspec(w=1), _rowspec(w=1),
                  _fullspec((F, F)), _fullspec((1, F))],
        out_specs=[_rowspec(), _rowspec(w=1)],
        out_shape=[jax.ShapeDtypeStruct((NP, F), jnp.float32),
                   jax.ShapeDtypeStruct((NP, 1), jnp.float32)],
    )(x, p0, p1, d0, d1, W, b)


def _tc_enc2(h1, q0, q1, inv, W, b, dTp):
    def body(h1_r, q0_r, q1_r, inv_r, W_r, b_r, dT_r, h_r, asg_r):
        agg = (q0_r[...] + q1_r[...]) * inv_r[...]
        h = jnp.maximum(
            jnp.dot(h1_r[...] + agg, W_r[...],
                    preferred_element_type=jnp.float32) + b_r[...], 0.0)
        h_r[...] = h
        lg = jnp.dot(h, dT_r[...], preferred_element_type=jnp.float32)
        col = lax.broadcasted_iota(jnp.int32, (BN, F), 1)
        lgm = jnp.where(col < A, lg, -1e30)
        m = jnp.max(lgm, axis=1, keepdims=True)
        asg_r[...] = jnp.min(jnp.where(lgm >= m, col, F), axis=1, keepdims=True)

    return pl.pallas_call(
        body,
        grid=(NBLK,),
        in_specs=[_rowspec(), _rowspec(), _rowspec(), _rowspec(w=1),
                  _fullspec((F, F)), _fullspec((1, F)), _fullspec((F, F))],
        out_specs=[_rowspec(), _rowspec(w=1)],
        out_shape=[jax.ShapeDtypeStruct((NP, F), jnp.float32),
                   jax.ShapeDtypeStruct((NP, 1), jnp.int32)],
    )(h1, q0, q1, inv, W, b, dTp)


def _tc_gconv(xin, r0, r1, Wself, Wk, b):
    def body(x_r, r0_r, r1_r, Ws_r, Wk_r, b_r, out_r):
        aggk = r0_r[...] + r1_r[...]
        Wk = Wk_r[...]
        ek = (jnp.dot(aggk, Wk[0], preferred_element_type=jnp.float32)
              + jnp.dot(aggk, Wk[1], preferred_element_type=jnp.float32)
              + jnp.dot(aggk, Wk[2], preferred_element_type=jnp.float32))
        out_r[...] = jnp.maximum(
            jnp.dot(x_r[...], Ws_r[...], preferred_element_type=jnp.float32)
            + ek * (1.0 / 3.0) + b_r[...], 0.0)

    return pl.pallas_call(
        body,
        grid=(NBLK,),
        in_specs=[_rowspec(), _rowspec(), _rowspec(),
                  _fullspec((F, F)), _fullspec((3, F, F)), _fullspec((1, F))],
        out_specs=_rowspec(),
        out_shape=jax.ShapeDtypeStruct((NP, F), jnp.float32),
    )(xin, r0, r1, Wself, Wk, b)


def _tc_pool1(x1, x2, x3, pp, awTp, batch2d):
    def body(x1_r, x2_r, x3_r, pp_r, aw_r, b_r, xc_r, gate_r, gm_r, mscr):
        i = pl.program_id(0)
        pp = pp_r[...]
        pn = pp * (1.0 / (jnp.sqrt(jnp.sum(pp * pp)) + 1e-9))
        x1v, x2v, x3v = x1_r[...], x2_r[...], x3_r[...]
        hi = lax.Precision.HIGHEST
        s1 = jnp.tanh(jnp.dot(x1v, pn, precision=hi,
                              preferred_element_type=jnp.float32))
        s2 = jnp.tanh(jnp.dot(x2v, pn, precision=hi,
                              preferred_element_type=jnp.float32))
        s3 = jnp.tanh(jnp.dot(x3v, pn, precision=hi,
                              preferred_element_type=jnp.float32))
        m = jnp.maximum(jnp.maximum(s1, s2), s3)
        e1 = jnp.exp(s1 - m)
        e2 = jnp.exp(s2 - m)
        e3 = jnp.exp(s3 - m)
        es = e1 + e2 + e3
        xc = (e1 * x1v + e2 * x2v + e3 * x3v) / es
        xc_r[...] = xc
        gate = jnp.dot(xc, aw_r[...], preferred_element_type=jnp.float32)
        row = lax.broadcasted_iota(jnp.int32, (BN, F), 0) + i * BN
        col = lax.broadcasted_iota(jnp.int32, (BN, F), 1)
        gate = jnp.where((row < N) & (col < H), gate, -1e30)
        gate_r[...] = gate

        @pl.when(i == 0)
        def _():
            mscr[...] = jnp.full((8, G), -1e30, jnp.float32)

        oh = b_r[...] == lax.broadcasted_iota(jnp.int32, (BN, G), 1)
        for h in range(H):
            tmp = jnp.where(oh, jnp.broadcast_to(gate[:, h:h + 1], (BN, G)),
                            -1e30)
            mh = jnp.max(tmp, axis=0, keepdims=True)
            mscr[h:h + 1, :] = jnp.maximum(mscr[h:h + 1, :], mh)
        gm_r[...] = mscr[...]

    return pl.pallas_call(
        body,
        grid=(NBLK,),
        in_specs=[_rowspec(), _rowspec(), _rowspec(),
                  _fullspec((F, 1)), _fullspec((F, F)), _rowspec(w=1)],
        out_specs=[_rowspec(), _rowspec(), pl.BlockSpec((8, G), lambda i: (0, 0))],
        out_shape=[jax.ShapeDtypeStruct((NP, F), jnp.float32),
                   jax.ShapeDtypeStruct((NP, F), jnp.float32),
                   jax.ShapeDtypeStruct((8, G), jnp.float32)],
        scratch_shapes=[pltpu.VMEM((8, G), jnp.float32)],
    )(x1, x2, x3, pp, awTp, batch2d)


def _tc_pool2(gate, gmaxp, batch2d):
    def body(gate_r, gm_r, b_r, eg_r, den_r, accden):
        i = pl.program_id(0)
        oh = (b_r[...] == lax.broadcasted_iota(jnp.int32, (BN, G), 1)
              ).astype(jnp.float32)
        gmb = jnp.dot(oh, gm_r[...], precision=lax.Precision.HIGHEST,
                      preferred_element_type=jnp.float32)
        eg = jnp.exp(gate_r[...] - gmb)
        eg_r[...] = eg

        @pl.when(i == 0)
        def _():
            accden[...] = jnp.zeros((G, F), jnp.float32)

        accden[...] += lax.dot_general(oh, eg, (((0,), (0,)), ((), ())),
                                       precision=lax.Precision.HIGHEST,
                                       preferred_element_type=jnp.float32)
        den_r[...] = accden[...]

    return pl.pallas_call(
        body,
        grid=(NBLK,),
        in_specs=[_rowspec(), pl.BlockSpec((G, F), lambda i: (0, 0)),
                  _rowspec(w=1)],
        out_specs=[_rowspec(), pl.BlockSpec((G, F), lambda i: (0, 0))],
        out_shape=[jax.ShapeDtypeStruct((NP, F), jnp.float32),
                   jax.ShapeDtypeStruct((G, F), jnp.float32)],
        scratch_shapes=[pltpu.VMEM((G, F), jnp.float32)],
    )(gate, gmaxp, batch2d)


def _tc_pool3(eg, den, xc, batch2d, W1, b1, W2p, b2p):
    def body(eg_r, den_r, xc_r, b_r, W1_r, b1_r, W2_r, b2_r,
             gf_r, lg_r, accgf):
        i = pl.program_id(0)
        oh = (b_r[...] == lax.broadcasted_iota(jnp.int32, (BN, G), 1)
              ).astype(jnp.float32)
        denb = jnp.dot(oh, den_r[...], precision=lax.Precision.HIGHEST,
                       preferred_element_type=jnp.float32)
        att = eg_r[...] / (denb + 1e-9)
        attbar = jnp.sum(att, axis=1, keepdims=True) * (1.0 / H)

        @pl.when(i == 0)
        def _():
            accgf[...] = jnp.zeros((G, F), jnp.float32)

        accgf[...] += lax.dot_general(oh, attbar * xc_r[...],
                                      (((0,), (0,)), ((), ())),
                                      precision=lax.Precision.HIGHEST,
                                      preferred_element_type=jnp.float32)

        @pl.when(i == NBLK - 1)
        def _():
            gf = accgf[...]
            gf_r[...] = gf
            z = jnp.maximum(
                jnp.dot(gf, W1_r[...], preferred_element_type=jnp.float32)
                + b1_r[...], 0.0)
            u = jnp.dot(z, W2_r[...], preferred_element_type=jnp.float32) \
                + b2_r[...]
            col = lax.broadcasted_iota(jnp.int32, (G, F), 1)
            um = jnp.where(col < NCLS, u, -1e30)
            m = jnp.max(um, axis=1, keepdims=True)
            esum = jnp.sum(jnp.where(col < NCLS, jnp.exp(um - m), 0.0),
                           axis=1, keepdims=True)
            lg_r[...] = jnp.where(col < NCLS, u - m - jnp.log(esum), 0.0)

    return pl.pallas_call(
        body,
        grid=(NBLK,),
        in_specs=[_rowspec(), pl.BlockSpec((G, F), lambda i: (0, 0)),
                  _rowspec(), _rowspec(w=1),
                  _fullspec((F, F)), _fullspec((1, F)),
                  _fullspec((F, F)), _fullspec((1, F))],
        out_specs=[pl.BlockSpec((G, F), lambda i: (0, 0)),
                   pl.BlockSpec((G, F), lambda i: (0, 0))],
        out_shape=[jax.ShapeDtypeStruct((G, F), jnp.float32),
                   jax.ShapeDtypeStruct((G, F), jnp.float32)],
        scratch_shapes=[pltpu.VMEM((G, F), jnp.float32)],
    )(eg, den, xc, batch2d, W1, b1, W2p, b2p)


def kernel(x, edge_index, batch, enc_W1, enc_b1, enc_W2, enc_b2, dictionary,
           g1_Wself, g1_Wk, g1_b, g2_Wself, g2_Wk, g2_b, g3_Wself, g3_Wk, g3_b,
           pool_p, att_w, cls_W1, cls_b1, cls_W2, cls_b2):
    f32 = jnp.float32
    x_ext = jnp.concatenate([x, jnp.zeros((NP - N, F), f32)], axis=0)
    padi = jnp.full((EP - E,), N, jnp.int32)
    srcp = jnp.concatenate([edge_index[0], padi])
    dstp = jnp.concatenate([edge_index[1], padi])
    z2d = jnp.zeros((ROWS_PER_SUB, F), f32)
    z1d = jnp.zeros((ROWS_PER_SUB,), f32)
    ones_c = jnp.ones((CHUNK,), f32)

    part, degp = _sc_agg_deg(x_ext, srcp, dstp, z2d, z1d, ones_c)
    d0 = degp[0].reshape(NP, 1)
    d1 = degp[1].reshape(NP, 1)
    h1, inv = _tc_enc1(x_ext, part[0], part[1], d0, d1,
                       enc_W1, enc_b1.reshape(1, F))

    q = _sc_agg(h1, srcp, dstp, z2d)
    dTp = jnp.zeros((F, F), f32).at[:, :A].set(dictionary.T)
    h2, asg = _tc_enc2(h1, q[0], q[1], inv, enc_W2, enc_b2.reshape(1, F), dTp)
    src2, dst2 = _sc_mask(asg.reshape(NP), srcp, dstp)

    r = _sc_agg(h2, src2, dst2, z2d)
    x1 = _tc_gconv(h2, r[0], r[1], g1_Wself, g1_Wk, g1_b.reshape(1, F))
    r = _sc_agg(x1, src2, dst2, z2d)
    x2 = _tc_gconv(x1, r[0], r[1], g2_Wself, g2_Wk, g2_b.reshape(1, F))
    r = _sc_agg(x2, src2, dst2, z2d)
    x3 = _tc_gconv(x2, r[0], r[1], g3_Wself, g3_Wk, g3_b.reshape(1, F))

    awTp = jnp.zeros((F, F), f32).at[:, :H].set(att_w.T)
    batch2d = jnp.concatenate([batch, jnp.full((NP - N,), G - 1, jnp.int32)]
                              ).reshape(NP, 1)
    xc, gate, gmax8 = _tc_pool1(x1, x2, x3, pool_p.reshape(F, 1), awTp, batch2d)

    gm4 = gmax8[:H]
    gm4 = jnp.where(gm4 > -1e29, gm4, 0.0)
    gmaxp = jnp.zeros((G, F), f32).at[:, :H].set(gm4.T)
    eg, den = _tc_pool2(gate, gmaxp, batch2d)

    W2p = jnp.zeros((F, F), f32).at[:, :NCLS].set(cls_W2)
    b2p = jnp.zeros((1, F), f32).at[:, :NCLS].set(cls_b2)
    gf, lgfull = _tc_pool3(eg, den, xc, batch2d,
                           cls_W1, cls_b1.reshape(1, F), W2p, b2p)
    return (lgfull[:, :NCLS], gf)


# speedup vs baseline: 9.0333x; 1.0002x over previous
import functools
import jax
import jax.numpy as jnp
from jax import lax
from jax.experimental import pallas as pl
from jax.experimental.pallas import tpu as pltpu, tpu_sc as plsc

N = 10000
E = 160000
F = 128
NP = 10240
EP = 163840
G = 64
H = 4
A = 16
NCLS = 10
CHUNK = 128
NSUB = 16
NCORE = 2
NWORK = NCORE * NSUB
CH_PER_W = EP // (CHUNK * NWORK)
ROWS_PER_SUB = NP // NSUB
BN = 1024
NBLK = NP // BN

def _get_mesh():
    return plsc.VectorSubcoreMesh(core_axis_name="c", subcore_axis_name="s",
                                  num_cores=NCORE, num_subcores=NSUB)


def _sc_agg_deg(x_hbm, srcp, dstp, z2d, z1d, ones_hbm):
    @functools.partial(
        pl.kernel,
        out_type=(jax.ShapeDtypeStruct((NCORE, NP, F), jnp.float32),
                  jax.ShapeDtypeStruct((NCORE, NP), jnp.float32)),
        mesh=_get_mesh(),
        scratch_types=[
            pltpu.VMEM((CHUNK,), jnp.int32),
            pltpu.VMEM((CHUNK,), jnp.int32),
            pltpu.VMEM((CHUNK, F), jnp.float32),
            pltpu.VMEM((CHUNK,), jnp.float32),
            pltpu.VMEM_SHARED((NP, F), jnp.float32),
            pltpu.VMEM_SHARED((NP,), jnp.float32),
            pltpu.SemaphoreType.DMA,
        ],
    )
    def k(x_r, src_r, dst_r, z2d_r, z1d_r, ones_r, out_r, outd_r,
          idx_s, idx_d, rows, ones_v, acc, accd, sem):
        c = lax.axis_index("c")
        s = lax.axis_index("s")
        w = s * NCORE + c
        r0 = s * ROWS_PER_SUB
        pltpu.sync_copy(z2d_r, acc.at[pl.ds(r0, ROWS_PER_SUB)])
        pltpu.sync_copy(z1d_r, accd.at[pl.ds(r0, ROWS_PER_SUB)])
        pltpu.sync_copy(ones_r, ones_v)
        plsc.subcore_barrier()
        base = w * (CH_PER_W * CHUNK)

        def step(t, _):
            off = base + t * CHUNK
            pltpu.sync_copy(src_r.at[pl.ds(off, CHUNK)], idx_s)
            pltpu.sync_copy(dst_r.at[pl.ds(off, CHUNK)], idx_d)
            pltpu.async_copy(x_r.at[idx_s], rows, sem).wait()
            pltpu.sync_copy(rows, acc.at[idx_d], add=True)
            pltpu.sync_copy(ones_v, accd.at[idx_d], add=True)
            return _

        lax.fori_loop(0, CH_PER_W, step, None)
        plsc.subcore_barrier()
        pltpu.sync_copy(acc.at[pl.ds(r0, ROWS_PER_SUB)],
                        out_r.at[c, pl.ds(r0, ROWS_PER_SUB)])
        pltpu.sync_copy(accd.at[pl.ds(r0, ROWS_PER_SUB)],
                        outd_r.at[c, pl.ds(r0, ROWS_PER_SUB)])

    return k(x_hbm, srcp, dstp, z2d, z1d, ones_hbm)


def _sc_agg(x_hbm, srcp, dstp, z2d):
    @functools.partial(
        pl.kernel,
        out_type=jax.ShapeDtypeStruct((NCORE, NP, F), jnp.float32),
        mesh=_get_mesh(),
        scratch_types=[
            pltpu.VMEM((CHUNK,), jnp.int32),
            pltpu.VMEM((CHUNK,), jnp.int32),
            pltpu.VMEM((CHUNK, F), jnp.float32),
            pltpu.VMEM_SHARED((NP, F), jnp.float32),
            pltpu.SemaphoreType.DMA,
        ],
    )
    def k(x_r, src_r, dst_r, z2d_r, out_r, idx_s, idx_d, rows, acc, sem):
        c = lax.axis_index("c")
        s = lax.axis_index("s")
        w = s * NCORE + c
        r0 = s * ROWS_PER_SUB
        pltpu.sync_copy(z2d_r, acc.at[pl.ds(r0, ROWS_PER_SUB)])
        plsc.subcore_barrier()
        base = w * (CH_PER_W * CHUNK)

        def step(t, _):
            off = base + t * CHUNK
            pltpu.sync_copy(src_r.at[pl.ds(off, CHUNK)], idx_s)
            pltpu.sync_copy(dst_r.at[pl.ds(off, CHUNK)], idx_d)
            pltpu.async_copy(x_r.at[idx_s], rows, sem).wait()
            pltpu.sync_copy(rows, acc.at[idx_d], add=True)
            return _

        lax.fori_loop(0, CH_PER_W, step, None)
        plsc.subcore_barrier()
        pltpu.sync_copy(acc.at[pl.ds(r0, ROWS_PER_SUB)],
                        out_r.at[c, pl.ds(r0, ROWS_PER_SUB)])

    return k(x_hbm, srcp, dstp, z2d)


def _sc_mask(assign, srcp, dstp):
    @functools.partial(
        pl.kernel,
        out_type=(jax.ShapeDtypeStruct((EP,), jnp.int32),
                  jax.ShapeDtypeStruct((EP,), jnp.int32)),
        mesh=_get_mesh(),
        scratch_types=[
            pltpu.VMEM((CHUNK,), jnp.int32),
            pltpu.VMEM((CHUNK,), jnp.int32),
            pltpu.VMEM((CHUNK,), jnp.int32),
            pltpu.VMEM((CHUNK,), jnp.int32),
            pltpu.VMEM((CHUNK,), jnp.int32),
            pltpu.VMEM((CHUNK,), jnp.int32),
            pltpu.SemaphoreType.DMA,
        ],
    )
    def k(asg_r, src_r, dst_r, src2_r, dst2_r, bs, bd, a1, a2, bs2, bd2, sem):
        c = lax.axis_index("c")
        s = lax.axis_index("s")
        w = s * NCORE + c
        base = w * (CH_PER_W * CHUNK)
        lane = jax.lax.iota(jnp.int32, 16)

        def step(t, _):
            off = base + t * CHUNK
            pltpu.sync_copy(src_r.at[pl.ds(off, CHUNK)], bs)
            pltpu.sync_copy(dst_r.at[pl.ds(off, CHUNK)], bd)
            cp1 = pltpu.async_copy(asg_r.at[bs], a1, sem)
            cp2 = pltpu.async_copy(asg_r.at[bd], a2, sem)
            cp1.wait()
            cp2.wait()
            for j in range(CHUNK // 16):
                si = bs[pl.ds(j * 16, 16)]
                di = bd[pl.ds(j * 16, 16)]
                eq = a1[pl.ds(j * 16, 16)] == a2[pl.ds(j * 16, 16)]
                dummy = N + jnp.remainder(off + j * 16 + lane, NP - N)
                bs2[pl.ds(j * 16, 16)] = jnp.where(eq, si, dummy)
                bd2[pl.ds(j * 16, 16)] = jnp.where(eq, di, dummy)
            pltpu.sync_copy(bs2, src2_r.at[pl.ds(off, CHUNK)])
            pltpu.sync_copy(bd2, dst2_r.at[pl.ds(off, CHUNK)])
            return _

        lax.fori_loop(0, CH_PER_W, step, None)

    return k(assign, srcp, dstp)


def _rowspec(bn=BN, w=F):
    return pl.BlockSpec((bn, w), lambda i: (i, 0))


def _fullspec(shape):
    nd = len(shape)
    return pl.BlockSpec(shape, lambda i: (0,) * nd)


def _tc_enc1(x, p0, p1, d0, d1, W, b):
    def body(x_r, p0_r, p1_r, d0_r, d1_r, W_r, b_r, h_r, inv_r):
        deg = jnp.maximum(d0_r[...] + d1_r[...], 1.0)
        inv = 1.0 / deg
        agg = (p0_r[...] + p1_r[...]) * inv
        h_r[...] = jnp.maximum(
            jnp.dot(x_r[...] + agg, W_r[...],
                    preferred_element_type=jnp.float32) + b_r[...], 0.0)
        inv_r[...] = inv

    return pl.pallas_call(
        body,
        grid=(NBLK,),
        in_specs=[_rowspec(), _rowspec(), _rowspec(),
                  _rowspec(w=1), _rowspec(w=1),
                  _fullspec((F, F)), _fullspec((1, F))],
        out_specs=[_rowspec(), _rowspec(w=1)],
        out_shape=[jax.ShapeDtypeStruct((NP, F), jnp.float32),
                   jax.ShapeDtypeStruct((NP, 1), jnp.float32)],
    )(x, p0, p1, d0, d1, W, b)


def _tc_enc2(h1, q0, q1, inv, W, b, dTp):
    def body(h1_r, q0_r, q1_r, inv_r, W_r, b_r, dT_r, h_r, asg_r):
        agg = (q0_r[...] + q1_r[...]) * inv_r[...]
        h = jnp.maximum(
            jnp.dot(h1_r[...] + agg, W_r[...],
                    preferred_element_type=jnp.float32) + b_r[...], 0.0)
        h_r[...] = h
        lg = jnp.dot(h, dT_r[...], preferred_element_type=jnp.float32)
        col = lax.broadcasted_iota(jnp.int32, (BN, F), 1)
        lgm = jnp.where(col < A, lg, -1e30)
        m = jnp.max(lgm, axis=1, keepdims=True)
        asg_r[...] = jnp.min(jnp.where(lgm >= m, col, F), axis=1, keepdims=True)

    return pl.pallas_call(
        body,
        grid=(NBLK,),
        in_specs=[_rowspec(), _rowspec(), _rowspec(), _rowspec(w=1),
                  _fullspec((F, F)), _fullspec((1, F)), _fullspec((F, F))],
        out_specs=[_rowspec(), _rowspec(w=1)],
        out_shape=[jax.ShapeDtypeStruct((NP, F), jnp.float32),
                   jax.ShapeDtypeStruct((NP, 1), jnp.int32)],
    )(h1, q0, q1, inv, W, b, dTp)


def _tc_gconv(xin, r0, r1, Wself, Wk, b):
    def body(x_r, r0_r, r1_r, Ws_r, Wk_r, b_r, out_r):
        aggk = r0_r[...] + r1_r[...]
        Wk = Wk_r[...]
        ek = (jnp.dot(aggk, Wk[0], preferred_element_type=jnp.float32)
              + jnp.dot(aggk, Wk[1], preferred_element_type=jnp.float32)
              + jnp.dot(aggk, Wk[2], preferred_element_type=jnp.float32))
        out_r[...] = jnp.maximum(
            jnp.dot(x_r[...], Ws_r[...], preferred_element_type=jnp.float32)
            + ek / 3.0 + b_r[...], 0.0)

    return pl.pallas_call(
        body,
        grid=(NBLK,),
        in_specs=[_rowspec(), _rowspec(), _rowspec(),
                  _fullspec((F, F)), _fullspec((3, F, F)), _fullspec((1, F))],
        out_specs=_rowspec(),
        out_shape=jax.ShapeDtypeStruct((NP, F), jnp.float32),
    )(xin, r0, r1, Wself, Wk, b)


def _tc_pool1(x1, x2, x3, pp, awTp, batch2d):
    def body(x1_r, x2_r, x3_r, pp_r, aw_r, b_r, xc_r, gate_r, gm_r, mscr):
        i = pl.program_id(0)
        pp = pp_r[...]
        pn = pp * (1.0 / (jnp.sqrt(jnp.sum(pp * pp)) + 1e-9))
        x1v, x2v, x3v = x1_r[...], x2_r[...], x3_r[...]
        hi = lax.Precision.HIGHEST
        s1 = jnp.tanh(jnp.dot(x1v, pn, precision=hi,
                              preferred_element_type=jnp.float32))
        s2 = jnp.tanh(jnp.dot(x2v, pn, precision=hi,
                              preferred_element_type=jnp.float32))
        s3 = jnp.tanh(jnp.dot(x3v, pn, precision=hi,
                              preferred_element_type=jnp.float32))
        m = jnp.maximum(jnp.maximum(s1, s2), s3)
        e1 = jnp.exp(s1 - m)
        e2 = jnp.exp(s2 - m)
        e3 = jnp.exp(s3 - m)
        es = e1 + e2 + e3
        xc = (e1 / es) * x1v + (e2 / es) * x2v + (e3 / es) * x3v
        xc_r[...] = xc
        gate = jnp.dot(xc, aw_r[...], preferred_element_type=jnp.float32)
        row = lax.broadcasted_iota(jnp.int32, (BN, F), 0) + i * BN
        col = lax.broadcasted_iota(jnp.int32, (BN, F), 1)
        gate = jnp.where((row < N) & (col < H), gate, -1e30)
        gate_r[...] = gate

        @pl.when(i == 0)
        def _():
            mscr[...] = jnp.full((8, G), -1e30, jnp.float32)

        oh = b_r[...] == lax.broadcasted_iota(jnp.int32, (BN, G), 1)
        for h in range(H):
            tmp = jnp.where(oh, jnp.broadcast_to(gate[:, h:h + 1], (BN, G)),
                            -1e30)
            mh = jnp.max(tmp, axis=0, keepdims=True)
            mscr[h:h + 1, :] = jnp.maximum(mscr[h:h + 1, :], mh)
        gm_r[...] = mscr[...]

    return pl.pallas_call(
        body,
        grid=(NBLK,),
        in_specs=[_rowspec(), _rowspec(), _rowspec(),
                  _fullspec((F, 1)), _fullspec((F, F)), _rowspec(w=1)],
        out_specs=[_rowspec(), _rowspec(), pl.BlockSpec((8, G), lambda i: (0, 0))],
        out_shape=[jax.ShapeDtypeStruct((NP, F), jnp.float32),
                   jax.ShapeDtypeStruct((NP, F), jnp.float32),
                   jax.ShapeDtypeStruct((8, G), jnp.float32)],
        scratch_shapes=[pltpu.VMEM((8, G), jnp.float32)],
    )(x1, x2, x3, pp, awTp, batch2d)


def _tc_pool2(gate, gmaxp, batch2d):
    def body(gate_r, gm_r, b_r, eg_r, den_r, accden):
        i = pl.program_id(0)
        oh = (b_r[...] == lax.broadcasted_iota(jnp.int32, (BN, G), 1)
              ).astype(jnp.float32)
        gmb = jnp.dot(oh, gm_r[...], precision=lax.Precision.HIGHEST,
                      preferred_element_type=jnp.float32)
        eg = jnp.exp(gate_r[...] - gmb)
        eg_r[...] = eg

        @pl.when(i == 0)
        def _():
            accden[...] = jnp.zeros((G, F), jnp.float32)

        accden[...] += lax.dot_general(oh, eg, (((0,), (0,)), ((), ())),
                                       precision=lax.Precision.HIGHEST,
                                       preferred_element_type=jnp.float32)
        den_r[...] = accden[...]

    return pl.pallas_call(
        body,
        grid=(NBLK,),
        in_specs=[_rowspec(), pl.BlockSpec((G, F), lambda i: (0, 0)),
                  _rowspec(w=1)],
        out_specs=[_rowspec(), pl.BlockSpec((G, F), lambda i: (0, 0))],
        out_shape=[jax.ShapeDtypeStruct((NP, F), jnp.float32),
                   jax.ShapeDtypeStruct((G, F), jnp.float32)],
        scratch_shapes=[pltpu.VMEM((G, F), jnp.float32)],
    )(gate, gmaxp, batch2d)


def _tc_pool3(eg, den, xc, batch2d, W1, b1, W2p, b2p):
    def body(eg_r, den_r, xc_r, b_r, W1_r, b1_r, W2_r, b2_r,
             gf_r, lg_r, accgf):
        i = pl.program_id(0)
        oh = (b_r[...] == lax.broadcasted_iota(jnp.int32, (BN, G), 1)
              ).astype(jnp.float32)
        denb = jnp.dot(oh, den_r[...], precision=lax.Precision.HIGHEST,
                       preferred_element_type=jnp.float32)
        att = eg_r[...] / (denb + 1e-9)
        attbar = jnp.sum(att, axis=1, keepdims=True) * (1.0 / H)

        @pl.when(i == 0)
        def _():
            accgf[...] = jnp.zeros((G, F), jnp.float32)

        accgf[...] += lax.dot_general(oh, attbar * xc_r[...],
                                      (((0,), (0,)), ((), ())),
                                      precision=lax.Precision.HIGHEST,
                                      preferred_element_type=jnp.float32)

        @pl.when(i == NBLK - 1)
        def _():
            gf = accgf[...]
            gf_r[...] = gf
            z = jnp.maximum(
                jnp.dot(gf, W1_r[...], preferred_element_type=jnp.float32)
                + b1_r[...], 0.0)
            u = jnp.dot(z, W2_r[...], preferred_element_type=jnp.float32) \
                + b2_r[...]
            col = lax.broadcasted_iota(jnp.int32, (G, F), 1)
            um = jnp.where(col < NCLS, u, -1e30)
            m = jnp.max(um, axis=1, keepdims=True)
            esum = jnp.sum(jnp.where(col < NCLS, jnp.exp(um - m), 0.0),
                           axis=1, keepdims=True)
            lg_r[...] = jnp.where(col < NCLS, u - m - jnp.log(esum), 0.0)

    return pl.pallas_call(
        body,
        grid=(NBLK,),
        in_specs=[_rowspec(), pl.BlockSpec((G, F), lambda i: (0, 0)),
                  _rowspec(), _rowspec(w=1),
                  _fullspec((F, F)), _fullspec((1, F)),
                  _fullspec((F, F)), _fullspec((1, F))],
        out_specs=[pl.BlockSpec((G, F), lambda i: (0, 0)),
                   pl.BlockSpec((G, F), lambda i: (0, 0))],
        out_shape=[jax.ShapeDtypeStruct((G, F), jnp.float32),
                   jax.ShapeDtypeStruct((G, F), jnp.float32)],
        scratch_shapes=[pltpu.VMEM((G, F), jnp.float32)],
    )(eg, den, xc, batch2d, W1, b1, W2p, b2p)


def kernel(x, edge_index, batch, enc_W1, enc_b1, enc_W2, enc_b2, dictionary,
           g1_Wself, g1_Wk, g1_b, g2_Wself, g2_Wk, g2_b, g3_Wself, g3_Wk, g3_b,
           pool_p, att_w, cls_W1, cls_b1, cls_W2, cls_b2):
    f32 = jnp.float32
    x_ext = jnp.concatenate([x, jnp.zeros((NP - N, F), f32)], axis=0)
    padi = jnp.full((EP - E,), N, jnp.int32)
    srcp = jnp.concatenate([edge_index[0], padi])
    dstp = jnp.concatenate([edge_index[1], padi])
    z2d = jnp.zeros((ROWS_PER_SUB, F), f32)
    z1d = jnp.zeros((ROWS_PER_SUB,), f32)
    ones_c = jnp.ones((CHUNK,), f32)

    part, degp = _sc_agg_deg(x_ext, srcp, dstp, z2d, z1d, ones_c)
    d0 = degp[0].reshape(NP, 1)
    d1 = degp[1].reshape(NP, 1)
    h1, inv = _tc_enc1(x_ext, part[0], part[1], d0, d1,
                       enc_W1, enc_b1.reshape(1, F))

    q = _sc_agg(h1, srcp, dstp, z2d)
    dTp = jnp.zeros((F, F), f32).at[:, :A].set(dictionary.T)
    h2, asg = _tc_enc2(h1, q[0], q[1], inv, enc_W2, enc_b2.reshape(1, F), dTp)
    src2, dst2 = _sc_mask(asg.reshape(NP), srcp, dstp)

    r = _sc_agg(h2, src2, dst2, z2d)
    x1 = _tc_gconv(h2, r[0], r[1], g1_Wself, g1_Wk, g1_b.reshape(1, F))
    r = _sc_agg(x1, src2, dst2, z2d)
    x2 = _tc_gconv(x1, r[0], r[1], g2_Wself, g2_Wk, g2_b.reshape(1, F))
    r = _sc_agg(x2, src2, dst2, z2d)
    x3 = _tc_gconv(x2, r[0], r[1], g3_Wself, g3_Wk, g3_b.reshape(1, F))

    awTp = jnp.zeros((F, F), f32).at[:, :H].set(att_w.T)
    batch2d = jnp.concatenate([batch, jnp.full((NP - N,), G - 1, jnp.int32)]
                              ).reshape(NP, 1)
    xc, gate, gmax8 = _tc_pool1(x1, x2, x3, pool_p.reshape(F, 1), awTp, batch2d)

    gm4 = gmax8[:H]
    gm4 = jnp.where(gm4 > -1e29, gm4, 0.0)
    gmaxp = jnp.zeros((G, F), f32).at[:, :H].set(gm4.T)
    eg, den = _tc_pool2(gate, gmaxp, batch2d)

    W2p = jnp.zeros((F, F), f32).at[:, :NCLS].set(cls_W2)
    b2p = jnp.zeros((1, F), f32).at[:, :NCLS].set(cls_b2)
    gf, lgfull = _tc_pool3(eg, den, xc, batch2d,
                           cls_W1, cls_b1.reshape(1, F), W2p, b2p)
    return (lgfull[:, :NCLS], gf)
